# Initial kernel scaffold; baseline (speedup 1.0000x reference)
import jax
import jax.numpy as jnp
from jax.experimental import pallas as pl


def kernel(input_ids, table):
    raise NotImplementedError("write your pallas kernel here")


# speedup vs baseline: 4.3108x; 4.3108x over previous
import functools

import jax
import jax.numpy as jnp
from jax import lax
from jax.experimental import pallas as pl
from jax.experimental.pallas import tpu as pltpu
from jax.experimental.pallas import tpu_sc as plsc

B, L, D = 16384, 200, 10
N = B * L
NC, NS, LANES = 2, 16, 16
NW = NC * NS
IDS_PER_W = N // NW
WAVE = 2048
NWAVES = IDS_PER_W // WAVE
GROUPS = WAVE // LANES


def _make_emb():
    mesh = plsc.VectorSubcoreMesh(core_axis_name="c", subcore_axis_name="s")

    @functools.partial(
        pl.kernel,
        mesh=mesh,
        out_type=jax.ShapeDtypeStruct((N * D,), jnp.float32),
        scratch_types=[
            pltpu.VMEM((100 * D,), jnp.float32),
            pltpu.VMEM((WAVE,), jnp.int32),
            pltpu.VMEM((WAVE * D,), jnp.float32),
        ],
        compiler_params=pltpu.CompilerParams(needs_layout_passes=False),
    )
    def emb(ids_hbm, table_hbm, out_hbm, table_v, idx_v, rows_v):
        wid = lax.axis_index("s") * NC + lax.axis_index("c")
        id0 = wid * IDS_PER_W
        pltpu.sync_copy(table_hbm, table_v)
        siota = lax.iota(jnp.int32, LANES) * D

        def wave_body(w, carry):
            base = id0 + w * WAVE
            pltpu.sync_copy(ids_hbm.at[pl.ds(base, WAVE)], idx_v)

            def group_body(g, carry2):
                ids = idx_v[pl.ds(g * LANES, LANES)]
                ebase = ids * D
                obase = siota + g * (LANES * D)
                for d in range(D):
                    col = plsc.load_gather(table_v, [ebase + d])
                    plsc.store_scatter(rows_v, [obase + d], col)
                return carry2

            lax.fori_loop(0, GROUPS, group_body, 0)
            pltpu.sync_copy(rows_v, out_hbm.at[pl.ds(base * D, WAVE * D)])
            return carry

        lax.fori_loop(0, NWAVES, wave_body, 0)

    return emb


_emb = _make_emb()


def kernel(input_ids, table):
    ids_flat = input_ids.reshape(N).astype(jnp.int32)
    out = _emb(ids_flat, table.reshape(100 * D))
    return out.reshape(B, L, D)


# speedup vs baseline: 5.9349x; 1.3767x over previous
import functools

import jax
import jax.numpy as jnp
from jax import lax
from jax.experimental import pallas as pl
from jax.experimental.pallas import tpu as pltpu
from jax.experimental.pallas import tpu_sc as plsc

B, L, D = 16384, 200, 10
N = B * L
NC, NS, LANES = 2, 16, 16
NW = NC * NS
IDS_PER_W = N // NW
WAVE = 512
NWAVES = IDS_PER_W // WAVE
GROUPS = WAVE // LANES


def _make_emb():
    mesh = plsc.VectorSubcoreMesh(core_axis_name="c", subcore_axis_name="s")

    @functools.partial(
        pl.kernel,
        mesh=mesh,
        out_type=jax.ShapeDtypeStruct((N, D), jnp.float32),
        scratch_types=[
            pltpu.VMEM((100 * D,), jnp.float32),
            pltpu.VMEM((WAVE,), jnp.int32),
            pltpu.VMEM((WAVE, D), jnp.float32),
        ],
        compiler_params=pltpu.CompilerParams(needs_layout_passes=False),
    )
    def emb(ids_hbm, table_hbm, out_hbm, table_v, idx_v, rows_v):
        wid = lax.axis_index("s") * NC + lax.axis_index("c")
        id0 = wid * IDS_PER_W
        pltpu.sync_copy(table_hbm, table_v)
        iota = lax.iota(jnp.int32, LANES)

        def wave_body(w, carry):
            base = id0 + w * WAVE
            pltpu.sync_copy(ids_hbm.at[pl.ds(base, WAVE)], idx_v)

            def group_body(g, carry2):
                ids = idx_v[pl.ds(g * LANES, LANES)]
                ebase = ids * D
                qvec = iota + g * LANES
                for d in range(D):
                    col = plsc.load_gather(table_v, [ebase + d])
                    dvec = jnp.full((LANES,), d, jnp.int32)
                    plsc.store_scatter(rows_v, [qvec, dvec], col)
                return carry2

            lax.fori_loop(0, GROUPS, group_body, 0)
            pltpu.sync_copy(rows_v, out_hbm.at[pl.ds(base, WAVE)])
            return carry

        lax.fori_loop(0, NWAVES, wave_body, 0)

    return emb


_emb = _make_emb()


def kernel(input_ids, table):
    ids_flat = input_ids.reshape(N).astype(jnp.int32)
    out = _emb(ids_flat, table.reshape(100 * D))
    return out.reshape(B, L, D)


# speedup vs baseline: 35.0522x; 5.9061x over previous
import functools

import jax
import jax.numpy as jnp
from jax import lax
from jax.experimental import pallas as pl
from jax.experimental.pallas import tpu as pltpu
from jax.experimental.pallas import tpu_sc as plsc

B, L, D = 16384, 200, 10
NC, NS, LANES = 2, 16, 16
NW = NC * NS
BW = B // NW
NL = 8
NWAVES = L // NL
GROUPS = BW // LANES


def _make_emb():
    mesh = plsc.VectorSubcoreMesh(core_axis_name="c", subcore_axis_name="s")

    @functools.partial(
        pl.kernel,
        mesh=mesh,
        out_type=jax.ShapeDtypeStruct((D * L, B), jnp.float32),
        scratch_types=[
            pltpu.VMEM((D, 100), jnp.float32),
            pltpu.VMEM((NL, BW), jnp.int32),
            pltpu.VMEM((D, NL, BW), jnp.float32),
        ],
        compiler_params=pltpu.CompilerParams(needs_layout_passes=False),
    )
    def emb(ids_hbm, tabt_hbm, out_hbm, tabt_v, idx_v, rows_v):
        wid = lax.axis_index("s") * NC + lax.axis_index("c")
        b0 = wid * BW
        pltpu.sync_copy(tabt_hbm, tabt_v)
        dsplat = [jnp.full((LANES,), d, jnp.int32) for d in range(D)]

        def wave_body(w, carry):
            l0 = w * NL
            pltpu.sync_copy(ids_hbm.at[pl.ds(l0, NL), pl.ds(b0, BW)], idx_v)

            def group_body(g, carry2):
                off = g * LANES
                for l in range(NL):
                    ids = idx_v[l, pl.ds(off, LANES)]
                    for d in range(D):
                        col = plsc.load_gather(tabt_v, [dsplat[d], ids])
                        rows_v[d, l, pl.ds(off, LANES)] = col
                return carry2

            lax.fori_loop(0, GROUPS, group_body, 0)
            for d in range(D):
                pltpu.sync_copy(
                    rows_v.at[d],
                    out_hbm.at[pl.ds(d * L + l0, NL), pl.ds(b0, BW)])
            return carry

        lax.fori_loop(0, NWAVES, wave_body, 0)

    return emb


_emb = _make_emb()


def kernel(input_ids, table):
    idsT = input_ids.T.astype(jnp.int32)
    tabT = table.T
    out2d = _emb(idsT, tabT)
    return out2d.reshape(D, L, B).transpose(2, 1, 0)


# speedup vs baseline: 183.2559x; 5.2281x over previous
import functools

import jax
import jax.numpy as jnp
from jax import lax
from jax.experimental import pallas as pl
from jax.experimental.pallas import tpu as pltpu
from jax.experimental.pallas import tpu_sc as plsc

B, L, D = 16384, 200, 10
NC, NS, LANES = 2, 16, 16
NW = NC * NS
BW = B // NW
NL = 8
NWAVES = L // NL
GROUPS = BW // LANES
NPAIRS = (NWAVES - 1) // 2


def _make_emb():
    mesh = plsc.VectorSubcoreMesh(core_axis_name="c", subcore_axis_name="s")

    @functools.partial(
        pl.kernel,
        mesh=mesh,
        out_type=jax.ShapeDtypeStruct((D, L, B), jnp.float32),
        scratch_types=[
            pltpu.VMEM((D, 100), jnp.float32),
            pltpu.VMEM((2, NL, BW), jnp.int32),
            pltpu.VMEM((2, D, NL, BW), jnp.float32),
            pltpu.SemaphoreType.DMA,
            pltpu.SemaphoreType.DMA,
            pltpu.SemaphoreType.DMA,
            pltpu.SemaphoreType.DMA,
        ],
        compiler_params=pltpu.CompilerParams(needs_layout_passes=False),
    )
    def emb(ids_hbm, tabt_hbm, out_hbm, tabt_v, idx_v, rows_v,
            in0, in1, out0, out1):
        wid = lax.axis_index("s") * NC + lax.axis_index("c")
        b0 = wid * BW
        pltpu.sync_copy(tabt_hbm, tabt_v)
        dsplat = [jnp.full((LANES,), d, jnp.int32) for d in range(D)]
        in_sems = [in0, in1]
        out_sems = [out0, out1]

        def in_dma(slot, w):
            return pltpu.make_async_copy(
                ids_hbm.at[pl.ds(w * NL, NL), pl.ds(b0, BW)],
                idx_v.at[slot], in_sems[slot])

        def out_dma(slot, w):
            return pltpu.make_async_copy(
                rows_v.at[slot],
                out_hbm.at[:, pl.ds(w * NL, NL), pl.ds(b0, BW)],
                out_sems[slot])

        def compute(slot):
            @functools.partial(plsc.parallel_loop, 0, GROUPS, unroll=2)
            def _(g):
                off = g * LANES
                for l in range(NL):
                    ids = idx_v[slot, l, pl.ds(off, LANES)]
                    for d in range(D):
                        col = plsc.load_gather(tabt_v, [dsplat[d], ids])
                        rows_v[slot, d, l, pl.ds(off, LANES)] = col

        def wave(slot, w, i, prefetch_slot, prefetch_w):
            in_dma(prefetch_slot, prefetch_w).start()
            in_dma(slot, w).wait()

            @pl.when(i > 0)
            def _():
                out_dma(slot, w - 2).wait()

            compute(slot)
            out_dma(slot, w).start()

        in_dma(0, 0).start()

        def pair_body(i, carry):
            w0 = 2 * i
            wave(0, w0, i, 1, w0 + 1)
            wave(1, w0 + 1, i, 0, w0 + 2)
            return carry

        lax.fori_loop(0, NPAIRS, pair_body, 0)

        last = NWAVES - 1
        in_dma(0, last).wait()
        out_dma(0, last - 2).wait()
        compute(0)
        out_dma(0, last).start()
        out_dma(1, last - 1).wait()
        out_dma(0, last).wait()

    return emb


_emb = _make_emb()


def kernel(input_ids, table):
    idsT = input_ids.T.astype(jnp.int32)
    tabT = table.T
    out = _emb(idsT, tabT)
    return out.transpose(2, 1, 0)
